# Initial kernel scaffold; baseline (speedup 1.0000x reference)
import jax
import jax.numpy as jnp
from jax.experimental import pallas as pl


def kernel(charges, species, batch_index, natoms, total_charge):
    raise NotImplementedError("write your pallas kernel here")


# speedup vs baseline: 20.1314x; 20.1314x over previous
import functools

import jax
import jax.numpy as jnp
import numpy as np
from jax import lax
from jax.experimental import pallas as pl
from jax.experimental.pallas import tpu as pltpu
from jax.experimental.pallas import tpu_sc as plsc

_Z = np.arange(87)
_D3_HARDNESSES = (0.15 + 0.35 * np.abs(np.sin(0.37 * _Z + 0.2))).astype(np.float32)
_D3_VDW_RADII = (1.2 + 2.5 * np.abs(np.cos(0.23 * _Z + 0.1))).astype(np.float32)
_ETA_TABLE_NP = (_D3_HARDNESSES + (2.0 / np.pi) ** 0.5 / _D3_VDW_RADII).astype(np.float32)

N_ATOMS = 100_000
NSYS = 1024

NW = 16
ROW = 128
KPW = 49
CPW = KPW * ROW
N_PAD = NW * CPW
NROWS = N_PAD // ROW
SEG_PAD = 1040
ETA_PAD = 96


def _sc_body(q_hbm, sp_hbm, b_hbm, tc_hbm, eta_hbm,
             qf_hbm, dq_hbm, ec_hbm,
             q2d, sp2d, b2d, s2d, eta2d, qf2d, ec2d,
             eta_tab, f_tab, zbuf, qtot_loc, ssum_loc, tc_loc, dq_buf,
             qtot_sh, ssum_sh, f_sh):
  w = lax.axis_index("s")

  @pl.when(w == 0)
  def _():
    zero = jnp.zeros((16,), jnp.float32)
    def zinit(g, _):
      zbuf[pl.ds(g * 16, 16)] = zero
      return ()
    lax.fori_loop(0, SEG_PAD // 16, zinit, ())
    pltpu.sync_copy(zbuf, qtot_sh)
    pltpu.sync_copy(zbuf, ssum_sh)

  row0 = w * KPW
  pltpu.sync_copy(q_hbm.at[pl.ds(row0, KPW)], q2d)
  pltpu.sync_copy(sp_hbm.at[pl.ds(row0, KPW)], sp2d)
  pltpu.sync_copy(b_hbm.at[pl.ds(row0, KPW)], b2d)
  pltpu.sync_copy(eta_hbm, eta_tab)

  def s_row(j, _):
    def s_grp(l, _):
      sp = sp2d[j, pl.ds(l * 16, 16)]
      eta = plsc.load_gather(eta_tab, [sp])
      s = 1.0 / (1e-6 + 2.0 * jnp.abs(eta))
      eta2d[j, pl.ds(l * 16, 16)] = eta
      s2d[j, pl.ds(l * 16, 16)] = s
      return ()
    lax.fori_loop(0, ROW // 16, s_grp, ())
    return ()
  lax.fori_loop(0, KPW, s_row, ())

  plsc.subcore_barrier()

  def sc_row(j, _):
    pltpu.sync_copy(q2d.at[j], qtot_sh.at[b2d.at[j]], add=True)
    pltpu.sync_copy(s2d.at[j], ssum_sh.at[b2d.at[j]], add=True)
    return ()
  lax.fori_loop(0, KPW, sc_row, ())

  plsc.subcore_barrier()

  @pl.when(w == 0)
  def _():
    pltpu.sync_copy(qtot_sh, qtot_loc)
    pltpu.sync_copy(ssum_sh, ssum_loc)
    pltpu.sync_copy(tc_hbm, tc_loc)
    def f_grp(g, _):
      sl = pl.ds(g * 16, 16)
      dqv = tc_loc[sl] - qtot_loc[sl]
      dq_buf[sl] = dqv
      qtot_loc[sl] = dqv / ssum_loc[sl]
      return ()
    lax.fori_loop(0, NSYS // 16, f_grp, ())
    zero = jnp.zeros((16,), jnp.float32)
    def f_pad(g, _):
      qtot_loc[pl.ds(NSYS + g * 16, 16)] = zero
      return ()
    lax.fori_loop(0, (SEG_PAD - NSYS) // 16, f_pad, ())
    pltpu.sync_copy(dq_buf, dq_hbm)
    pltpu.sync_copy(qtot_loc, f_sh)

  plsc.subcore_barrier()

  pltpu.sync_copy(f_sh, f_tab)
  def o_row(j, _):
    def o_grp(l, _):
      sl = pl.ds(l * 16, 16)
      b = b2d[j, sl]
      fv = plsc.load_gather(f_tab, [b])
      dqa = s2d[j, sl] * fv
      qf2d[j, sl] = q2d[j, sl] + dqa
      ec2d[j, sl] = 0.5 * eta2d[j, sl] * dqa * dqa
      return ()
    lax.fori_loop(0, ROW // 16, o_grp, ())
    return ()
  lax.fori_loop(0, KPW, o_row, ())

  pltpu.sync_copy(qf2d, qf_hbm.at[pl.ds(row0, KPW)])
  pltpu.sync_copy(ec2d, ec_hbm.at[pl.ds(row0, KPW)])


@functools.partial(jax.jit, static_argnames=())
def _run(q_pad, sp_pad, b_pad, total_charge, eta_tab):
  mesh = plsc.VectorSubcoreMesh(
      core_axis_name="c", subcore_axis_name="s", num_cores=1)
  f32 = jnp.float32
  kern = pl.kernel(
      _sc_body,
      out_type=(
          jax.ShapeDtypeStruct((NROWS, ROW), f32),
          jax.ShapeDtypeStruct((NSYS,), f32),
          jax.ShapeDtypeStruct((NROWS, ROW), f32),
      ),
      mesh=mesh,
      compiler_params=pltpu.CompilerParams(
          use_tc_tiling_on_sc=False, needs_layout_passes=False),
      scratch_types=(
          pltpu.VMEM((KPW, ROW), f32),
          pltpu.VMEM((KPW, ROW), jnp.int32),
          pltpu.VMEM((KPW, ROW), jnp.int32),
          pltpu.VMEM((KPW, ROW), f32),
          pltpu.VMEM((KPW, ROW), f32),
          pltpu.VMEM((KPW, ROW), f32),
          pltpu.VMEM((KPW, ROW), f32),
          pltpu.VMEM((ETA_PAD,), f32),
          pltpu.VMEM((SEG_PAD,), f32),
          pltpu.VMEM((SEG_PAD,), f32),
          pltpu.VMEM((SEG_PAD,), f32),
          pltpu.VMEM((SEG_PAD,), f32),
          pltpu.VMEM((NSYS,), f32),
          pltpu.VMEM((NSYS,), f32),
          pltpu.VMEM_SHARED((SEG_PAD,), f32),
          pltpu.VMEM_SHARED((SEG_PAD,), f32),
          pltpu.VMEM_SHARED((SEG_PAD,), f32),
      ),
  )
  return kern(q_pad, sp_pad, b_pad, total_charge, eta_tab)


def kernel(charges, species, batch_index, natoms, total_charge):
  del natoms
  n = charges.shape[0]
  pad = N_PAD - n
  q_pad = jnp.concatenate(
      [charges, jnp.zeros((pad,), charges.dtype)]).reshape(NROWS, ROW)
  sp_pad = jnp.concatenate(
      [species.astype(jnp.int32), jnp.zeros((pad,), jnp.int32)]
  ).reshape(NROWS, ROW)
  b_pad = jnp.concatenate(
      [batch_index.astype(jnp.int32),
       jnp.full((pad,), NSYS, jnp.int32)]).reshape(NROWS, ROW)
  eta_tab = jnp.concatenate(
      [jnp.asarray(_ETA_TABLE_NP),
       jnp.zeros((ETA_PAD - _ETA_TABLE_NP.shape[0],), jnp.float32)])
  qf2d, dq, ec2d = _run(q_pad, sp_pad, b_pad,
                        total_charge.astype(jnp.float32), eta_tab)
  qf = qf2d.reshape(-1)[:n]
  ec = ec2d.reshape(-1)[:n]
  return qf, dq, ec
